# Initial kernel scaffold; baseline (speedup 1.0000x reference)
import jax
import jax.numpy as jnp
from jax.experimental import pallas as pl


def kernel(query, key):
    raise NotImplementedError("write your pallas kernel here")


# speedup vs baseline: 9.4818x; 9.4818x over previous
import jax
import jax.numpy as jnp
from jax.experimental import pallas as pl
from jax.experimental.pallas import tpu as pltpu

QK_DIM = 32
TOPK = 16
N = 2048
BATCH = 8
BQ = 256


def _topk_route_kernel(q_ref, k_ref, out_ref):
    scale = QK_DIM ** (-0.5)
    q = q_ref[0] * jnp.float32(scale)
    k = k_ref[0]
    logits = jax.lax.dot_general(
        q, k, (((1,), (1,)), ((), ())),
        preferred_element_type=jnp.float32)
    col = jax.lax.broadcasted_iota(jnp.int32, logits.shape, 1)
    neg = jnp.float32(-jnp.inf)
    idxs = []
    for _ in range(TOPK):
        m = jnp.max(logits, axis=1, keepdims=True)
        is_max = logits == m
        idx = jnp.min(jnp.where(is_max, col, N), axis=1, keepdims=True)
        idxs.append(idx)
        logits = jnp.where(col == idx, neg, logits)
    out_ref[0] = jnp.concatenate(idxs, axis=1)


def kernel(query, key):
    grid = (BATCH, N // BQ)
    return pl.pallas_call(
        _topk_route_kernel,
        grid=grid,
        in_specs=[
            pl.BlockSpec((1, BQ, QK_DIM), lambda b, i: (b, i, 0)),
            pl.BlockSpec((1, N, QK_DIM), lambda b, i: (b, 0, 0)),
        ],
        out_specs=pl.BlockSpec((1, BQ, TOPK), lambda b, i: (b, i, 0)),
        out_shape=jax.ShapeDtypeStruct((BATCH, N, TOPK), jnp.int32),
        compiler_params=pltpu.CompilerParams(
            dimension_semantics=("parallel", "parallel")),
    )(query, key)


# speedup vs baseline: 11.8180x; 1.2464x over previous
import jax
import jax.numpy as jnp
from jax.experimental import pallas as pl
from jax.experimental.pallas import tpu as pltpu

QK_DIM = 32
TOPK = 16
N = 2048
BATCH = 8
BQ = 256
LANES = 128
STACKS = 4
DEPTH = 4

_SORT4 = [(0, 1), (2, 3), (0, 2), (1, 3), (1, 2)]


def _topk_route_kernel(q_ref, k_ref, out_ref):
    scale = QK_DIM ** (-0.5)
    q = q_ref[0] * jnp.float32(scale)
    k = k_ref[0]
    logits = jax.lax.dot_general(
        q, k, (((1,), (1,)), ((), ())),
        preferred_element_type=jnp.float32)

    lane = jax.lax.broadcasted_iota(jnp.int32, (BQ, LANES), 1)
    neg = jnp.float32(-jnp.inf)

    sv, si = [], []
    for g in range(STACKS):
        vals = [logits[:, (g * DEPTH + d) * LANES:(g * DEPTH + d + 1) * LANES]
                for d in range(DEPTH)]
        idxs = [lane + (g * DEPTH + d) * LANES for d in range(DEPTH)]
        for a, b in _SORT4:
            ge = vals[a] >= vals[b]
            va = jnp.where(ge, vals[a], vals[b])
            vb = jnp.where(ge, vals[b], vals[a])
            ia = jnp.where(ge, idxs[a], idxs[b])
            ib = jnp.where(ge, idxs[b], idxs[a])
            vals[a], vals[b], idxs[a], idxs[b] = va, vb, ia, ib
        sv.append(vals)
        si.append(idxs)

    outs = []
    for _ in range(TOPK):
        tvs = [sv[g][0] for g in range(STACKS)]
        tis = [si[g][0] for g in range(STACKS)]
        while len(tvs) > 1:
            nvs, nis = [], []
            for p in range(0, len(tvs), 2):
                ge = tvs[p] >= tvs[p + 1]
                nvs.append(jnp.where(ge, tvs[p], tvs[p + 1]))
                nis.append(jnp.where(ge, tis[p], tis[p + 1]))
            tvs, tis = nvs, nis
        tv, ti = tvs[0], tis[0]
        wl = jnp.argmax(tv, axis=1, keepdims=True)
        widx = jnp.sum(jnp.where(lane == wl, ti, 0), axis=1,
                       keepdims=True)
        outs.append(widx)
        for g in range(STACKS):
            mask = si[g][0] == widx
            for d in range(DEPTH - 1):
                sv[g][d] = jnp.where(mask, sv[g][d + 1], sv[g][d])
                si[g][d] = jnp.where(mask, si[g][d + 1], si[g][d])
            sv[g][DEPTH - 1] = jnp.where(mask, neg, sv[g][DEPTH - 1])

    out_ref[0] = jnp.concatenate(outs, axis=1)


def kernel(query, key):
    grid = (BATCH, N // BQ)
    return pl.pallas_call(
        _topk_route_kernel,
        grid=grid,
        in_specs=[
            pl.BlockSpec((1, BQ, QK_DIM), lambda b, i: (b, i, 0)),
            pl.BlockSpec((1, N, QK_DIM), lambda b, i: (b, 0, 0)),
        ],
        out_specs=pl.BlockSpec((1, BQ, TOPK), lambda b, i: (b, i, 0)),
        out_shape=jax.ShapeDtypeStruct((BATCH, N, TOPK), jnp.int32),
        compiler_params=pltpu.CompilerParams(
            dimension_semantics=("parallel", "parallel")),
    )(query, key)


# speedup vs baseline: 12.5317x; 1.0604x over previous
import jax
import jax.numpy as jnp
from jax.experimental import pallas as pl
from jax.experimental.pallas import tpu as pltpu

QK_DIM = 32
TOPK = 16
N = 2048
BATCH = 8
BQ = 256
LANES = 128
STACKS = 4
DEPTH = 4

_SORT4 = [(0, 1), (2, 3), (0, 2), (1, 3), (1, 2)]


def _topk_route_kernel(q_ref, k_ref, out_ref):
    scale = QK_DIM ** (-0.5)
    q = q_ref[0] * jnp.float32(scale)
    k = k_ref[0]
    logits = jax.lax.dot_general(
        q, k, (((1,), (1,)), ((), ())),
        preferred_element_type=jnp.float32)

    col = jax.lax.broadcasted_iota(jnp.int32, logits.shape, 1)
    neg = jnp.float32(-jnp.inf)
    outs = []
    for _ in range(TOPK):
        idx = jnp.argmax(logits, axis=1, keepdims=True)
        outs.append(idx)
        logits = jnp.where(col == idx, neg, logits)

    out_ref[0] = jnp.concatenate(outs, axis=1)


def kernel(query, key):
    grid = (BATCH, N // BQ)
    return pl.pallas_call(
        _topk_route_kernel,
        grid=grid,
        in_specs=[
            pl.BlockSpec((1, BQ, QK_DIM), lambda b, i: (b, i, 0)),
            pl.BlockSpec((1, N, QK_DIM), lambda b, i: (b, 0, 0)),
        ],
        out_specs=pl.BlockSpec((1, BQ, TOPK), lambda b, i: (b, i, 0)),
        out_shape=jax.ShapeDtypeStruct((BATCH, N, TOPK), jnp.int32),
        compiler_params=pltpu.CompilerParams(
            dimension_semantics=("parallel", "parallel")),
    )(query, key)
